# baseline (speedup 1.0000x reference)
import functools

import jax
import jax.numpy as jnp
from jax import lax
from jax.experimental import pallas as pl
from jax.experimental.pallas import tpu as pltpu
from jax.experimental.pallas import tpu_sc as plsc


@functools.lru_cache(maxsize=None)
def _make_gather(V, D, S, B4):
    B = S * B4
    info = plsc.get_sparse_core_info()
    NC, NS = info.num_cores, info.num_subcores
    NW = NC * NS
    assert B % NW == 0
    b_per_w = B // NW
    CH = 32
    NBUF = 3
    GDEPTH = 2
    assert b_per_w % CH == 0
    n_chunks = b_per_w // CH
    HEAD = n_chunks % NBUF
    mesh = plsc.VectorSubcoreMesh(core_axis_name="c", subcore_axis_name="s")

    @functools.partial(
        pl.kernel,
        mesh=mesh,
        out_type=jax.ShapeDtypeStruct((S, B4, D), jnp.float32),
        scratch_types=[
            pltpu.VMEM((b_per_w,), jnp.int32),
        ]
        + [pltpu.VMEM((CH, D), jnp.float32)] * NBUF
        + [pltpu.SemaphoreType.DMA] * (2 * NBUF),
    )
    def k(idx_hbm, table_hbm, out3d, idx_v, *bufsem):
        bufs = bufsem[:NBUF]
        gs = bufsem[NBUF : 2 * NBUF]
        ss = bufsem[2 * NBUF :]
        out_hbm = out3d.reshape(B, D)
        wid = lax.axis_index("s") * NC + lax.axis_index("c")
        base = wid * b_per_w
        pltpu.sync_copy(idx_hbm.at[pl.ds(base, b_per_w)], idx_v)

        def start_g(c, j):
            off = pl.multiple_of(c * CH, 8)
            pltpu.async_copy(table_hbm.at[idx_v.at[pl.ds(off, CH)]], bufs[j], gs[j])

        def wait_g(j):
            pltpu.make_async_copy(
                table_hbm.at[idx_v.at[pl.ds(0, CH)]], bufs[j], gs[j]
            ).wait()

        def start_s(c, j):
            off = pl.multiple_of(c * CH, 8)
            pltpu.async_copy(bufs[j], out_hbm.at[pl.ds(base + off, CH)], ss[j])

        def wait_s(j):
            pltpu.make_async_copy(bufs[j], out_hbm.at[pl.ds(base, CH)], ss[j]).wait()

        for c0 in range(GDEPTH):
            start_g(c0, c0 % NBUF)

        def chunk_body(c, b):
            wait_g(b)
            start_s(c, b)
            bn = (b + GDEPTH) % NBUF
            if isinstance(c, int):
                if c >= NBUF - GDEPTH:
                    wait_s(bn)
                if c + GDEPTH < n_chunks:
                    start_g(c + GDEPTH, bn)
            else:
                pl.when(c >= NBUF - GDEPTH)(lambda: wait_s(bn))
                pl.when(c + GDEPTH < n_chunks)(lambda: start_g(c + GDEPTH, bn))

        for c0 in range(HEAD):
            chunk_body(c0, c0 % NBUF)

        def body(g, carry):
            for j in range(NBUF):
                chunk_body(HEAD + g * NBUF + j, (HEAD + j) % NBUF)
            return carry

        lax.fori_loop(0, (n_chunks - HEAD) // NBUF, body, 0)
        for c in range(n_chunks - (NBUF - GDEPTH), n_chunks):
            wait_s(c % NBUF)

    return k


def kernel(i, encoding):
    s, b = i.shape
    V, D = encoding.shape
    flat = i.reshape(-1).astype(jnp.int32)
    return _make_gather(V, D, s, b)(flat, encoding)


# speedup vs baseline: 1.0112x; 1.0112x over previous
import functools

import jax
import jax.numpy as jnp
from jax import lax
from jax.experimental import pallas as pl
from jax.experimental.pallas import tpu as pltpu
from jax.experimental.pallas import tpu_sc as plsc


@functools.lru_cache(maxsize=None)
def _make_gather(V, D, S, B4):
    B = S * B4
    info = plsc.get_sparse_core_info()
    NC, NS = info.num_cores, info.num_subcores
    NW = NC * NS
    assert B % NW == 0
    b_per_w = B // NW
    CH = 16
    NBUF = 7
    GDEPTH = 5
    assert b_per_w % CH == 0
    n_chunks = b_per_w // CH
    HEAD = n_chunks % NBUF
    mesh = plsc.VectorSubcoreMesh(core_axis_name="c", subcore_axis_name="s")

    @functools.partial(
        pl.kernel,
        mesh=mesh,
        out_type=jax.ShapeDtypeStruct((S, B4, D), jnp.float32),
        scratch_types=[
            pltpu.VMEM((b_per_w,), jnp.int32),
        ]
        + [pltpu.VMEM((CH, D), jnp.float32)] * NBUF
        + [pltpu.SemaphoreType.DMA] * (2 * NBUF),
    )
    def k(idx_hbm, table_hbm, out3d, idx_v, *bufsem):
        bufs = bufsem[:NBUF]
        gs = bufsem[NBUF : 2 * NBUF]
        ss = bufsem[2 * NBUF :]
        out_hbm = out3d.reshape(B, D)
        wid = lax.axis_index("s") * NC + lax.axis_index("c")
        base = wid * b_per_w
        pltpu.sync_copy(idx_hbm.at[pl.ds(base, b_per_w)], idx_v)

        def start_g(c, j):
            off = pl.multiple_of(c * CH, 8)
            pltpu.async_copy(table_hbm.at[idx_v.at[pl.ds(off, CH)]], bufs[j], gs[j])

        def wait_g(j):
            pltpu.make_async_copy(
                table_hbm.at[idx_v.at[pl.ds(0, CH)]], bufs[j], gs[j]
            ).wait()

        def start_s(c, j):
            off = pl.multiple_of(c * CH, 8)
            pltpu.async_copy(bufs[j], out_hbm.at[pl.ds(base + off, CH)], ss[j])

        def wait_s(j):
            pltpu.make_async_copy(bufs[j], out_hbm.at[pl.ds(base, CH)], ss[j]).wait()

        for c0 in range(GDEPTH):
            start_g(c0, c0 % NBUF)

        def chunk_body(c, b):
            wait_g(b)
            start_s(c, b)
            bn = (b + GDEPTH) % NBUF
            if isinstance(c, int):
                if c >= NBUF - GDEPTH:
                    wait_s(bn)
                if c + GDEPTH < n_chunks:
                    start_g(c + GDEPTH, bn)
            else:
                pl.when(c >= NBUF - GDEPTH)(lambda: wait_s(bn))
                pl.when(c + GDEPTH < n_chunks)(lambda: start_g(c + GDEPTH, bn))

        for c0 in range(HEAD):
            chunk_body(c0, c0 % NBUF)

        def body(g, carry):
            for j in range(NBUF):
                chunk_body(HEAD + g * NBUF + j, (HEAD + j) % NBUF)
            return carry

        lax.fori_loop(0, (n_chunks - HEAD) // NBUF, body, 0)
        for c in range(n_chunks - (NBUF - GDEPTH), n_chunks):
            wait_s(c % NBUF)

    return k


def kernel(i, encoding):
    s, b = i.shape
    V, D = encoding.shape
    flat = i.reshape(-1).astype(jnp.int32)
    return _make_gather(V, D, s, b)(flat, encoding)
